# Initial kernel scaffold; baseline (speedup 1.0000x reference)
import jax
import jax.numpy as jnp
from jax.experimental import pallas as pl


def kernel(company_features, daily_news_features, W_src, W_dst, att_src, att_dst, bias):
    raise NotImplementedError("write your pallas kernel here")


# speedup vs baseline: 13.5551x; 13.5551x over previous
import functools

import jax
import jax.numpy as jnp
from jax.experimental import pallas as pl
from jax.experimental.pallas import tpu as pltpu

N = 15
NP = 16
D = 512
H = 3
L = 3
NEG = -1e30


def _layer_kernel(x0_ref, news_ref, ws_ref, wd_ref, as_ref, ad_ref, b_ref,
                  out_ref):
    i = pl.program_id(0)

    x = jnp.where(i == 0, x0_ref[...], out_ref[...])

    ws0 = ws_ref[0, 0]
    w_avg = (ws0[:, :D] + ws0[:, D:2 * D] + ws0[:, 2 * D:]) * (1.0 / 3.0)
    acc = jnp.dot(news_ref[0], w_avg, preferred_element_type=jnp.float32)

    src_ok = jax.lax.broadcasted_iota(jnp.int32, (NP, NP), 0) < N

    for r in (1, 2):
        hs = jnp.dot(x, ws_ref[0, r], preferred_element_type=jnp.float32)
        hd = jnp.dot(x, wd_ref[0, r - 1], preferred_element_type=jnp.float32)
        a_s = as_ref[0, r - 1]
        a_d = ad_ref[0, r - 1]
        for h in range(H):
            hs_h = hs[:, h * D:(h + 1) * D]
            hd_h = hd[:, h * D:(h + 1) * D]
            al_s = jnp.sum(hs_h * a_s[h][None, :], axis=1, keepdims=True)
            al_d = jnp.sum(hd_h * a_d[h][None, :], axis=1, keepdims=True)
            alpha = al_s + jnp.transpose(al_d)
            alpha = jnp.where(alpha > 0, alpha, 0.2 * alpha)
            alpha = jnp.where(src_ok, alpha, NEG)
            amax = jnp.max(alpha, axis=0, keepdims=True)
            e = jnp.exp(alpha - amax)
            denom = jnp.sum(e, axis=0, keepdims=True)
            att = e / (denom + 1e-16)
            acc = acc + (1.0 / H) * jax.lax.dot_general(
                att, hs_h, (((0,), (0,)), ((), ())),
                preferred_element_type=jnp.float32)

    b = b_ref[0]
    acc = acc + (b[0] + b[1] + b[2])[None, :]
    out_ref[...] = jax.nn.sigmoid(acc * (1.0 / 3.0))


@jax.jit
def kernel(company_features, daily_news_features, W_src, W_dst, att_src,
           att_dst, bias):
    x0 = jnp.zeros((NP, D), jnp.float32).at[:N].set(company_features)
    news = jnp.zeros((L, NP, D), jnp.float32).at[:, :N].set(
        daily_news_features)
    wd = W_dst[:, 1:]
    a_s = att_src[:, 1:]
    a_d = att_dst[:, 1:]

    out = pl.pallas_call(
        _layer_kernel,
        grid=(L,),
        in_specs=[
            pl.BlockSpec((NP, D), lambda i: (0, 0)),
            pl.BlockSpec((1, NP, D), lambda i: (i, 0, 0)),
            pl.BlockSpec((1, 3, D, H * D), lambda i: (i, 0, 0, 0)),
            pl.BlockSpec((1, 2, D, H * D), lambda i: (i, 0, 0, 0)),
            pl.BlockSpec((1, 2, H, D), lambda i: (i, 0, 0, 0)),
            pl.BlockSpec((1, 2, H, D), lambda i: (i, 0, 0, 0)),
            pl.BlockSpec((1, 3, D), lambda i: (i, 0, 0)),
        ],
        out_specs=pl.BlockSpec((NP, D), lambda i: (0, 0)),
        out_shape=jax.ShapeDtypeStruct((NP, D), jnp.float32),
        compiler_params=pltpu.CompilerParams(
            dimension_semantics=("arbitrary",)),
    )(x0, news, W_src, wd, a_s, a_d, bias)
    return out[:N]


# speedup vs baseline: 23.3070x; 1.7194x over previous
import functools

import jax
import jax.numpy as jnp
from jax.experimental import pallas as pl
from jax.experimental.pallas import tpu as pltpu

N = 15
NP = 16
D = 512
H = 3
L = 3
NEG = -1e30


def _layer_kernel(x0_ref, news_ref, ws0_ref, ws1_ref, ws2_ref, wd1_ref,
                  wd2_ref, as_ref, ad_ref, b_ref, out_ref):
    i = pl.program_id(0)

    x = jnp.where(i == 0, x0_ref[...], out_ref[...])

    ws0 = ws0_ref[0, 0]
    w_avg = (ws0[:, :D] + ws0[:, D:2 * D] + ws0[:, 2 * D:]) * (1.0 / 3.0)
    acc = jnp.dot(news_ref[0], w_avg, preferred_element_type=jnp.float32)

    src_ok = jax.lax.broadcasted_iota(jnp.int32, (NP, NP), 0) < N

    for r in (1, 2):
        ws_r = ws1_ref if r == 1 else ws2_ref
        wd_r = wd1_ref if r == 1 else wd2_ref
        hs = jnp.dot(x, ws_r[0, 0], preferred_element_type=jnp.float32)
        hd = jnp.dot(x, wd_r[0, 0], preferred_element_type=jnp.float32)
        a_s = as_ref[0, r]
        a_d = ad_ref[0, r]
        for h in range(H):
            hs_h = hs[:, h * D:(h + 1) * D]
            hd_h = hd[:, h * D:(h + 1) * D]
            al_s = jnp.sum(hs_h * a_s[h][None, :], axis=1, keepdims=True)
            al_d = jnp.sum(hd_h * a_d[h][None, :], axis=1, keepdims=True)
            alpha = al_s + jnp.transpose(al_d)
            alpha = jnp.where(alpha > 0, alpha, 0.2 * alpha)
            alpha = jnp.where(src_ok, alpha, NEG)
            amax = jnp.max(alpha, axis=0, keepdims=True)
            e = jnp.exp(alpha - amax)
            denom = jnp.sum(e, axis=0, keepdims=True)
            att = e / (denom + 1e-16)
            acc = acc + (1.0 / H) * jax.lax.dot_general(
                att, hs_h, (((0,), (0,)), ((), ())),
                preferred_element_type=jnp.float32)

    b = b_ref[0]
    acc = acc + (b[0] + b[1] + b[2])[None, :]
    out_ref[...] = jax.nn.sigmoid(acc * (1.0 / 3.0))


@jax.jit
def kernel(company_features, daily_news_features, W_src, W_dst, att_src,
           att_dst, bias):
    x0 = jnp.zeros((NP, D), jnp.float32).at[:N].set(company_features)
    news = jnp.zeros((L, NP, D), jnp.float32).at[:, :N].set(
        daily_news_features)

    def wspec(r):
        return pl.BlockSpec((1, 1, D, H * D), lambda i, _r=r: (i, _r, 0, 0))

    out = pl.pallas_call(
        _layer_kernel,
        grid=(L,),
        in_specs=[
            pl.BlockSpec((NP, D), lambda i: (0, 0)),
            pl.BlockSpec((1, NP, D), lambda i: (i, 0, 0)),
            wspec(0), wspec(1), wspec(2),
            wspec(1), wspec(2),
            pl.BlockSpec((1, 3, H, D), lambda i: (i, 0, 0, 0)),
            pl.BlockSpec((1, 3, H, D), lambda i: (i, 0, 0, 0)),
            pl.BlockSpec((1, 3, D), lambda i: (i, 0, 0)),
        ],
        out_specs=pl.BlockSpec((NP, D), lambda i: (0, 0)),
        out_shape=jax.ShapeDtypeStruct((NP, D), jnp.float32),
        compiler_params=pltpu.CompilerParams(
            dimension_semantics=("arbitrary",)),
    )(x0, news, W_src, W_src, W_src, W_dst, W_dst, att_src, att_dst, bias)
    return out[:N]
